# baseline (speedup 1.0000x reference)
import functools

import jax
import jax.numpy as jnp
import numpy as np
from jax import lax
from jax.experimental import pallas as pl
from jax.experimental.pallas import tpu as pltpu
from jax.experimental.pallas import tpu_sc as plsc

N, E, D, H = 10000, 320000, 128, 128

NC, NS, L = 2, 16, 16
NW = NC * NS
C = 80
NCH = E // C
CPW = NCH // NW
REM = NCH % NW
NB = 3
NSLOT = -(-(CPW + (1 if REM else 0)) // NB) * NB
RPT = N // NS
ZROWS = 25


_PERM = np.empty((H,), np.int32)
for _f in range(H // 32):
    for _i in range(16):
        _PERM[32 * _f + 2 * _i] = 32 * _f + _i
        _PERM[32 * _f + 2 * _i + 1] = 32 * _f + 16 + _i


def _sc_scatter_body(y_hbm, pk_hbm, out_hbm,
                     pk_v, rows_bf, srows, zbuf, acc,
                     gs0, gs1, gs2, ss0, ss1, ss2):
    c = lax.axis_index("c")
    s = lax.axis_index("s")
    wid = c * NS + s
    nch = CPW + jnp.where(wid < REM, 1, 0)
    gsems = (gs0, gs1, gs2)
    ssems = (ss0, ss1, ss2)

    def load_idx(k, b):
        base = (wid + NW * k) * C
        pltpu.sync_copy(pk_hbm.at[:, pl.ds(base, C)], pk_v.at[b])

    def start_gather(b):
        pltpu.async_copy(y_hbm.at[pk_v.at[b, 0]], rows_bf.at[b], gsems[b])

    for k0 in range(2):
        load_idx(jnp.int32(k0), k0)
        start_gather(k0)

    zero16 = jnp.zeros((L,), jnp.float32)

    def zrow(i, carry):
        for j in range(H // L):
            zbuf[i, pl.ds(j * L, L)] = zero16
        return carry

    lax.fori_loop(0, ZROWS, zrow, 0)
    for t in range(RPT // ZROWS):
        pltpu.sync_copy(zbuf, acc.at[pl.ds(s * RPT + t * ZROWS, ZROWS)])
    plsc.subcore_barrier()

    def group(j2, carry):
        for b in range(NB):
            k = j2 * NB + b

            @pl.when(k < nch)
            def _process():
                pltpu.make_async_copy(y_hbm.at[pk_v.at[b, 0]],
                                      rows_bf.at[b], gsems[b]).wait()

                def _grp(g, carry2):
                    for u in range(4):
                        r = g * 4 + u
                        wbits = plsc.load_gather(
                            pk_v, [jnp.full((L,), b, jnp.int32),
                                   jnp.full((L,), 2, jnp.int32),
                                   jnp.full((L,), r, jnp.int32)])
                        wv = plsc.bitcast(wbits, jnp.float32)
                        for j in range(H // 32):
                            blk = plsc.bitcast(
                                rows_bf[b, r, pl.ds(32 * j, 32)], jnp.int32)
                            lo = plsc.bitcast(blk << 16, jnp.float32)
                            hi = plsc.bitcast(
                                blk & jnp.int32(-65536), jnp.float32)
                            srows[b, r, pl.ds(32 * j, L)] = lo * wv
                            srows[b, r, pl.ds(32 * j + L, L)] = hi * wv
                    return carry2

                lax.fori_loop(0, C // 4, _grp, 0)

                pltpu.async_copy(srows.at[b], acc.at[pk_v.at[b, 1]],
                                 ssems[b], add=True)

            @pl.when(k + 2 < nch)
            def _prefetch():
                bp = (b + 2) % NB

                @pl.when(k >= 1)
                def _wait_prev_scatter():
                    pltpu.make_async_copy(srows.at[bp],
                                          acc.at[pk_v.at[bp, 1]],
                                          ssems[bp]).wait()

                load_idx(k + 2, bp)
                start_gather(bp)
        return carry

    lax.fori_loop(0, NSLOT // NB, group, 0)

    for b in range(NB):
        pltpu.make_async_copy(srows.at[b], acc.at[pk_v.at[b, 1]],
                              ssems[b]).wait()
    plsc.subcore_barrier()
    pltpu.sync_copy(acc.at[pl.ds(s * RPT, RPT)],
                    out_hbm.at[c, pl.ds(s * RPT, RPT)])


@functools.lru_cache(maxsize=1)
def _sc_scatter_fn():
    mesh = plsc.VectorSubcoreMesh(core_axis_name="c", subcore_axis_name="s")
    return pl.kernel(
        _sc_scatter_body,
        mesh=mesh,
        compiler_params=pltpu.CompilerParams(use_tc_tiling_on_sc=False,
                                             needs_layout_passes=False),
        out_type=jax.ShapeDtypeStruct((NC, N, H), jnp.float32),
        scratch_types=[
            pltpu.VMEM((NB, 3, C), jnp.int32),
            pltpu.VMEM((NB, C, H), jnp.bfloat16),
            pltpu.VMEM((NB, C, H), jnp.float32),
            pltpu.VMEM((ZROWS, H), jnp.float32),
            pltpu.VMEM_SHARED((N, H), jnp.float32),
            pltpu.SemaphoreType.DMA,
            pltpu.SemaphoreType.DMA,
            pltpu.SemaphoreType.DMA,
            pltpu.SemaphoreType.DMA,
            pltpu.SemaphoreType.DMA,
            pltpu.SemaphoreType.DMA,
        ],
    )


_BN = 1000


def _tc_in_body(x_ref, w1_ref, o_ref):
    y = lax.dot_general(
        x_ref[...], w1_ref[...], (((1,), (1,)), ((), ())),
        preferred_element_type=jnp.float32)
    o_ref[...] = y.astype(jnp.bfloat16)


def _tc_mid_body(p_ref, w2_ref, w1n_ref, o_ref):
    t = jnp.maximum(p_ref[0] + p_ref[1], 0.0)
    h = lax.dot_general(t, w2_ref[...], (((1,), (1,)), ((), ())),
                        preferred_element_type=jnp.float32)
    y = lax.dot_general(h, w1n_ref[...], (((1,), (1,)), ((), ())),
                        preferred_element_type=jnp.float32)
    o_ref[...] = y.astype(jnp.bfloat16)


def _tc_out_body(p_ref, w2_ref, o_ref):
    t = jnp.maximum(p_ref[0] + p_ref[1], 0.0)
    z = lax.dot_general(t, w2_ref[...], (((1,), (1,)), ((), ())),
                        preferred_element_type=jnp.float32)
    nrm = jnp.sqrt(jnp.sum(z * z, axis=1, keepdims=True))
    o_ref[...] = z / jnp.maximum(nrm, 1e-12)


def _tc_in(x, W1):
    return pl.pallas_call(
        _tc_in_body,
        grid=(N // _BN,),
        in_specs=[pl.BlockSpec((_BN, D), lambda i: (i, 0)),
                  pl.BlockSpec((H, D), lambda i: (0, 0))],
        out_specs=pl.BlockSpec((_BN, H), lambda i: (i, 0)),
        out_shape=jax.ShapeDtypeStruct((N, H), jnp.bfloat16),
    )(x, W1)


def _tc_mid(p, W2, W1n):
    return pl.pallas_call(
        _tc_mid_body,
        grid=(N // _BN,),
        in_specs=[pl.BlockSpec((NC, _BN, H), lambda i: (0, i, 0)),
                  pl.BlockSpec((H, H), lambda i: (0, 0)),
                  pl.BlockSpec((H, H), lambda i: (0, 0))],
        out_specs=pl.BlockSpec((_BN, H), lambda i: (i, 0)),
        out_shape=jax.ShapeDtypeStruct((N, H), jnp.bfloat16),
    )(p, W2, W1n)


def _tc_out(p, W2):
    return pl.pallas_call(
        _tc_out_body,
        grid=(N // _BN,),
        in_specs=[pl.BlockSpec((NC, _BN, H), lambda i: (0, i, 0)),
                  pl.BlockSpec((H, H), lambda i: (0, 0))],
        out_specs=pl.BlockSpec((_BN, H), lambda i: (i, 0)),
        out_shape=jax.ShapeDtypeStruct((N, H), jnp.float32),
    )(p, W2)


def kernel(x, edge_index, edge_weight, W1_0, W2_0, W1_1, W2_1):
    wbits = lax.bitcast_convert_type(edge_weight, jnp.int32)
    pk = jnp.concatenate([edge_index, wbits[None]], axis=0)
    perm = jnp.asarray(_PERM)
    sc_scatter = _sc_scatter_fn()
    y0 = _tc_in(x, W1_0[perm])
    p0 = sc_scatter(y0, pk)
    y1 = _tc_mid(p0, W2_0, W1_1[perm])
    p1 = sc_scatter(y1, pk)
    return _tc_out(p1, W2_1)


# speedup vs baseline: 1.9139x; 1.9139x over previous
import functools

import jax
import jax.numpy as jnp
import numpy as np
from jax import lax
from jax.experimental import pallas as pl
from jax.experimental.pallas import tpu as pltpu
from jax.experimental.pallas import tpu_sc as plsc

N, E, D, H = 10000, 320000, 128, 128

NC, NS, L = 2, 16, 16
NW = NC * NS
C = 128
NCH = E // C
CPW = NCH // NW
REM = NCH % NW
NB = 3
NSLOT = -(-(CPW + (1 if REM else 0)) // NB) * NB
RPT = N // NS
ZROWS = 125


def _sc_scatter_body(y_hbm, pk_hbm, out_hbm,
                     pk_v, rows_v, acc,
                     gs0, gs1, gs2, ss0, ss1, ss2):
    c = lax.axis_index("c")
    s = lax.axis_index("s")
    wid = c * NS + s
    nch = CPW + jnp.where(wid < REM, 1, 0)
    gsems = (gs0, gs1, gs2)
    ssems = (ss0, ss1, ss2)

    def load_idx(k, b):
        base = (wid + NW * k) * C
        pltpu.sync_copy(pk_hbm.at[:, pl.ds(base, C)], pk_v.at[b])

    def start_gather(b):
        pltpu.async_copy(y_hbm.at[pk_v.at[b, 0]], rows_v.at[b], gsems[b])

    zero16 = jnp.zeros((L,), jnp.float32)

    def zrow(i, carry):
        for j in range(H // L):
            rows_v[0, i, pl.ds(j * L, L)] = zero16
        return carry

    lax.fori_loop(0, ZROWS, zrow, 0)
    for t in range(RPT // ZROWS):
        pltpu.sync_copy(rows_v.at[0, pl.ds(0, ZROWS)],
                        acc.at[pl.ds(s * RPT + t * ZROWS, ZROWS)])

    for k0 in range(2):
        load_idx(jnp.int32(k0), k0)
        start_gather(k0)
    plsc.subcore_barrier()

    def group(j2, carry):
        for b in range(NB):
            k = j2 * NB + b

            @pl.when(k < nch)
            def _process():
                pltpu.make_async_copy(y_hbm.at[pk_v.at[b, 0]],
                                      rows_v.at[b], gsems[b]).wait()

                def _grp(g, carry2):
                    for u in range(4):
                        r = g * 4 + u
                        wbits = plsc.load_gather(
                            pk_v, [jnp.full((L,), b, jnp.int32),
                                   jnp.full((L,), 2, jnp.int32),
                                   jnp.full((L,), r, jnp.int32)])
                        wv = plsc.bitcast(wbits, jnp.float32)
                        for j in range(H // L):
                            rows_v[b, r, pl.ds(j * L, L)] = (
                                rows_v[b, r, pl.ds(j * L, L)] * wv)
                    return carry2

                lax.fori_loop(0, C // 4, _grp, 0)

                pltpu.async_copy(rows_v.at[b], acc.at[pk_v.at[b, 1]],
                                 ssems[b], add=True)

            @pl.when(k + 2 < nch)
            def _prefetch():
                bp = (b + 2) % NB

                @pl.when(k >= 1)
                def _wait_prev_scatter():
                    pltpu.make_async_copy(rows_v.at[bp],
                                          acc.at[pk_v.at[bp, 1]],
                                          ssems[bp]).wait()

                load_idx(k + 2, bp)
                start_gather(bp)
        return carry

    lax.fori_loop(0, NSLOT // NB, group, 0)

    for b in range(NB):
        pltpu.make_async_copy(rows_v.at[b], acc.at[pk_v.at[b, 1]],
                              ssems[b]).wait()
    plsc.subcore_barrier()
    pltpu.sync_copy(acc.at[pl.ds(s * RPT, RPT)],
                    out_hbm.at[c, pl.ds(s * RPT, RPT)])


@functools.lru_cache(maxsize=1)
def _sc_scatter_fn():
    mesh = plsc.VectorSubcoreMesh(core_axis_name="c", subcore_axis_name="s")
    return pl.kernel(
        _sc_scatter_body,
        mesh=mesh,
        compiler_params=pltpu.CompilerParams(use_tc_tiling_on_sc=False,
                                             needs_layout_passes=False),
        out_type=jax.ShapeDtypeStruct((NC, N, H), jnp.float32),
        scratch_types=[
            pltpu.VMEM((NB, 3, C), jnp.int32),
            pltpu.VMEM((NB, C, H), jnp.float32),
            pltpu.VMEM_SHARED((N, H), jnp.float32),
            pltpu.SemaphoreType.DMA,
            pltpu.SemaphoreType.DMA,
            pltpu.SemaphoreType.DMA,
            pltpu.SemaphoreType.DMA,
            pltpu.SemaphoreType.DMA,
            pltpu.SemaphoreType.DMA,
        ],
    )


_BN = 1000


def _tc_in_body(x_ref, w1_ref, o_ref):
    o_ref[...] = lax.dot_general(
        x_ref[...], w1_ref[...], (((1,), (1,)), ((), ())),
        preferred_element_type=jnp.float32)


def _tc_mid_body(p_ref, w2_ref, w1n_ref, o_ref):
    t = jnp.maximum(p_ref[0] + p_ref[1], 0.0)
    h = lax.dot_general(t, w2_ref[...], (((1,), (1,)), ((), ())),
                        preferred_element_type=jnp.float32)
    o_ref[...] = lax.dot_general(h, w1n_ref[...], (((1,), (1,)), ((), ())),
                                 preferred_element_type=jnp.float32)


def _tc_out_body(p_ref, w2_ref, o_ref):
    t = jnp.maximum(p_ref[0] + p_ref[1], 0.0)
    z = lax.dot_general(t, w2_ref[...], (((1,), (1,)), ((), ())),
                        preferred_element_type=jnp.float32)
    nrm = jnp.sqrt(jnp.sum(z * z, axis=1, keepdims=True))
    o_ref[...] = z / jnp.maximum(nrm, 1e-12)


def _tc_in(x, W1):
    return pl.pallas_call(
        _tc_in_body,
        grid=(N // _BN,),
        in_specs=[pl.BlockSpec((_BN, D), lambda i: (i, 0)),
                  pl.BlockSpec((H, D), lambda i: (0, 0))],
        out_specs=pl.BlockSpec((_BN, H), lambda i: (i, 0)),
        out_shape=jax.ShapeDtypeStruct((N, H), jnp.float32),
    )(x, W1)


def _tc_mid(p, W2, W1n):
    return pl.pallas_call(
        _tc_mid_body,
        grid=(N // _BN,),
        in_specs=[pl.BlockSpec((NC, _BN, H), lambda i: (0, i, 0)),
                  pl.BlockSpec((H, H), lambda i: (0, 0)),
                  pl.BlockSpec((H, H), lambda i: (0, 0))],
        out_specs=pl.BlockSpec((_BN, H), lambda i: (i, 0)),
        out_shape=jax.ShapeDtypeStruct((N, H), jnp.float32),
    )(p, W2, W1n)


def _tc_out(p, W2):
    return pl.pallas_call(
        _tc_out_body,
        grid=(N // _BN,),
        in_specs=[pl.BlockSpec((NC, _BN, H), lambda i: (0, i, 0)),
                  pl.BlockSpec((H, H), lambda i: (0, 0))],
        out_specs=pl.BlockSpec((_BN, H), lambda i: (i, 0)),
        out_shape=jax.ShapeDtypeStruct((N, H), jnp.float32),
    )(p, W2)


def kernel(x, edge_index, edge_weight, W1_0, W2_0, W1_1, W2_1):
    wbits = lax.bitcast_convert_type(edge_weight, jnp.int32)
    pk = jnp.concatenate([edge_index, wbits[None]], axis=0)
    sc_scatter = _sc_scatter_fn()
    y0 = _tc_in(x, W1_0)
    p0 = sc_scatter(y0, pk)
    y1 = _tc_mid(p0, W2_0, W1_1)
    p1 = sc_scatter(y1, pk)
    return _tc_out(p1, W2_1)


# speedup vs baseline: 2.1347x; 1.1154x over previous
import functools

import jax
import jax.numpy as jnp
import numpy as np
from jax import lax
from jax.experimental import pallas as pl
from jax.experimental.pallas import tpu as pltpu
from jax.experimental.pallas import tpu_sc as plsc

N, E, D, H = 10000, 320000, 128, 128

NC, NS, L = 2, 16, 16
NW = NC * NS
C = 128
NCH = E // C
CPW = NCH // NW
REM = NCH % NW
NB = 3
NI = 4
NSU = 12
NSLOT = -(-(CPW + (1 if REM else 0)) // NSU) * NSU
RPT = N // NS
ZROWS = 125


def _sc_scatter_body(y_hbm, pk_hbm, out_hbm,
                     pk_v, rows_v, acc,
                     gs0, gs1, gs2, ss0, ss1, ss2, is0, is1, is2, is3):
    c = lax.axis_index("c")
    s = lax.axis_index("s")
    wid = c * NS + s
    nch = CPW + jnp.where(wid < REM, 1, 0)
    gsems = (gs0, gs1, gs2)
    ssems = (ss0, ss1, ss2)
    isems = (is0, is1, is2, is3)

    def load_idx(k, b4):
        base = (wid + NW * k) * C
        pltpu.sync_copy(pk_hbm.at[:, pl.ds(base, C)], pk_v.at[b4])

    def start_load_idx(k, b4):
        base = (wid + NW * k) * C
        pltpu.async_copy(pk_hbm.at[:, pl.ds(base, C)], pk_v.at[b4],
                         isems[b4])

    def start_gather(k4, b):
        pltpu.async_copy(y_hbm.at[pk_v.at[k4, 0]], rows_v.at[b], gsems[b])

    zero16 = jnp.zeros((L,), jnp.float32)

    def zrow(i, carry):
        for j in range(H // L):
            rows_v[0, i, pl.ds(j * L, L)] = zero16
        return carry

    lax.fori_loop(0, ZROWS, zrow, 0)
    for t in range(RPT // ZROWS):
        pltpu.sync_copy(rows_v.at[0, pl.ds(0, ZROWS)],
                        acc.at[pl.ds(s * RPT + t * ZROWS, ZROWS)])

    for k0 in range(3):
        load_idx(jnp.int32(k0), k0)
    for k0 in range(2):
        start_gather(k0, k0)
    plsc.subcore_barrier()

    def group(j12, carry):
        for b in range(NSU):
            k = j12 * NSU + b
            br = b % NB
            bi = b % NI

            @pl.when(k < nch)
            def _process():
                pltpu.make_async_copy(y_hbm.at[pk_v.at[bi, 0]],
                                      rows_v.at[br], gsems[br]).wait()

                def _grp(g, carry2):
                    for u in range(4):
                        r = g * 4 + u
                        wbits = plsc.load_gather(
                            pk_v, [jnp.full((L,), bi, jnp.int32),
                                   jnp.full((L,), 2, jnp.int32),
                                   jnp.full((L,), r, jnp.int32)])
                        wv = plsc.bitcast(wbits, jnp.float32)
                        for j in range(H // L):
                            rows_v[br, r, pl.ds(j * L, L)] = (
                                rows_v[br, r, pl.ds(j * L, L)] * wv)
                    return carry2

                lax.fori_loop(0, C // 4, _grp, 0)

                pltpu.async_copy(rows_v.at[br], acc.at[pk_v.at[bi, 1]],
                                 ssems[br], add=True)

            @pl.when(k + 2 < nch)
            def _prefetch():
                bpr = (b + 2) % NB
                bpi = (b + 2) % NI
                bqi = (b + 3) % NI

                @pl.when(k >= 1)
                def _wait_prev():
                    pltpu.make_async_copy(rows_v.at[bpr],
                                          acc.at[pk_v.at[bqi, 1]],
                                          ssems[bpr]).wait()
                    pltpu.make_async_copy(
                        pk_hbm.at[:, pl.ds((wid + NW * (k + 2)) * C, C)],
                        pk_v.at[bpi], isems[bpi]).wait()

                start_gather(bpi, bpr)

                @pl.when(k + 3 < nch)
                def _next_idx():
                    start_load_idx(k + 3, bqi)
        return carry

    lax.fori_loop(0, NSLOT // NSU, group, 0)

    for b in range(NB):
        pltpu.make_async_copy(rows_v.at[b], acc.at[pk_v.at[0, 1]],
                              ssems[b]).wait()
    plsc.subcore_barrier()
    pltpu.sync_copy(acc.at[pl.ds(s * RPT, RPT)],
                    out_hbm.at[c, pl.ds(s * RPT, RPT)])


@functools.lru_cache(maxsize=1)
def _sc_scatter_fn():
    mesh = plsc.VectorSubcoreMesh(core_axis_name="c", subcore_axis_name="s")
    return pl.kernel(
        _sc_scatter_body,
        mesh=mesh,
        compiler_params=pltpu.CompilerParams(use_tc_tiling_on_sc=False,
                                             needs_layout_passes=False),
        out_type=jax.ShapeDtypeStruct((NC, N, H), jnp.float32),
        scratch_types=[
            pltpu.VMEM((NI, 3, C), jnp.int32),
            pltpu.VMEM((NB, C, H), jnp.float32),
            pltpu.VMEM_SHARED((N, H), jnp.float32),
        ] + [pltpu.SemaphoreType.DMA] * (NB + NB + NI),
    )


_BN = 1000


def _tc_in_body(x_ref, w1_ref, o_ref):
    o_ref[...] = lax.dot_general(
        x_ref[...], w1_ref[...], (((1,), (1,)), ((), ())),
        preferred_element_type=jnp.float32)


def _tc_mid_body(p_ref, w2_ref, w1n_ref, o_ref):
    t = jnp.maximum(p_ref[0] + p_ref[1], 0.0)
    h = lax.dot_general(t, w2_ref[...], (((1,), (1,)), ((), ())),
                        preferred_element_type=jnp.float32)
    o_ref[...] = lax.dot_general(h, w1n_ref[...], (((1,), (1,)), ((), ())),
                                 preferred_element_type=jnp.float32)


def _tc_out_body(p_ref, w2_ref, o_ref):
    t = jnp.maximum(p_ref[0] + p_ref[1], 0.0)
    z = lax.dot_general(t, w2_ref[...], (((1,), (1,)), ((), ())),
                        preferred_element_type=jnp.float32)
    nrm = jnp.sqrt(jnp.sum(z * z, axis=1, keepdims=True))
    o_ref[...] = z / jnp.maximum(nrm, 1e-12)


def _tc_in(x, W1):
    return pl.pallas_call(
        _tc_in_body,
        grid=(N // _BN,),
        in_specs=[pl.BlockSpec((_BN, D), lambda i: (i, 0)),
                  pl.BlockSpec((H, D), lambda i: (0, 0))],
        out_specs=pl.BlockSpec((_BN, H), lambda i: (i, 0)),
        out_shape=jax.ShapeDtypeStruct((N, H), jnp.float32),
    )(x, W1)


def _tc_mid(p, W2, W1n):
    return pl.pallas_call(
        _tc_mid_body,
        grid=(N // _BN,),
        in_specs=[pl.BlockSpec((NC, _BN, H), lambda i: (0, i, 0)),
                  pl.BlockSpec((H, H), lambda i: (0, 0)),
                  pl.BlockSpec((H, H), lambda i: (0, 0))],
        out_specs=pl.BlockSpec((_BN, H), lambda i: (i, 0)),
        out_shape=jax.ShapeDtypeStruct((N, H), jnp.float32),
    )(p, W2, W1n)


def _tc_out(p, W2):
    return pl.pallas_call(
        _tc_out_body,
        grid=(N // _BN,),
        in_specs=[pl.BlockSpec((NC, _BN, H), lambda i: (0, i, 0)),
                  pl.BlockSpec((H, H), lambda i: (0, 0))],
        out_specs=pl.BlockSpec((_BN, H), lambda i: (i, 0)),
        out_shape=jax.ShapeDtypeStruct((N, H), jnp.float32),
    )(p, W2)


def kernel(x, edge_index, edge_weight, W1_0, W2_0, W1_1, W2_1):
    wbits = lax.bitcast_convert_type(edge_weight, jnp.int32)
    pk = jnp.concatenate([edge_index, wbits[None]], axis=0)
    sc_scatter = _sc_scatter_fn()
    y0 = _tc_in(x, W1_0)
    p0 = sc_scatter(y0, pk)
    y1 = _tc_mid(p0, W2_0, W1_1)
    p1 = sc_scatter(y1, pk)
    return _tc_out(p1, W2_1)
